# baseline (speedup 1.0000x reference)
import jax
import jax.numpy as jnp
from jax import lax
from jax.experimental import pallas as pl
from jax.experimental.pallas import tpu as pltpu

_NC = 80
_MAX_DET = 300
_NMS_IOU = 0.7
_CONF_THRES = 0.001
_LANES = 128
_ROWS = 264
_APAD = _ROWS * _LANES
_NK = 384


def _detect_kernel(x_ref, o_ref, cls_ref):
    sc = x_ref[0, 4:, :, :]
    smax = jnp.max(sc, axis=0)
    ci = lax.broadcasted_iota(jnp.int32, (_NC, _ROWS, _LANES), 0)
    cls = jnp.min(jnp.where(sc == smax[None], ci, jnp.int32(2147483647)), axis=0)
    cls_ref[...] = cls.astype(jnp.float32)

    fi = (lax.broadcasted_iota(jnp.int32, (_ROWS, _LANES), 0) * _LANES
          + lax.broadcasted_iota(jnp.int32, (_ROWS, _LANES), 1))
    lT = lax.broadcasted_iota(jnp.int32, (1, _NK), 1)
    ohl = lax.broadcasted_iota(jnp.int32, (1, _LANES), 1)

    zrow = jnp.zeros((1, _NK), jnp.float32)

    def topk_body(i, carry):
        s, cxT, cyT, wT, hT, cfT, clT = carry
        mx = jnp.max(s)
        idx = jnp.min(jnp.where(s == mx, fi, jnp.int32(2147483647)))
        r = idx // _LANES
        c = idx % _LANES
        oh = ohl == c
        row0 = x_ref[0, 0, pl.dslice(r, 1), :]
        row1 = x_ref[0, 1, pl.dslice(r, 1), :]
        row2 = x_ref[0, 2, pl.dslice(r, 1), :]
        row3 = x_ref[0, 3, pl.dslice(r, 1), :]
        rowc = cls_ref[pl.dslice(r, 1), :]
        cx = jnp.sum(jnp.where(oh, row0, 0.0))
        cy = jnp.sum(jnp.where(oh, row1, 0.0))
        w = jnp.sum(jnp.where(oh, row2, 0.0))
        h = jnp.sum(jnp.where(oh, row3, 0.0))
        cl = jnp.sum(jnp.where(oh, rowc, 0.0))
        upd = lT == i
        cxT = jnp.where(upd, cx, cxT)
        cyT = jnp.where(upd, cy, cyT)
        wT = jnp.where(upd, w, wT)
        hT = jnp.where(upd, h, hT)
        cfT = jnp.where(upd, mx, cfT)
        clT = jnp.where(upd, cl, clT)
        s = jnp.where(fi == idx, -2.0, s)
        return s, cxT, cyT, wT, hT, cfT, clT

    carry0 = (smax, zrow, zrow, zrow, zrow, zrow, zrow)
    _, cxT, cyT, wT, hT, cfT, clT = lax.fori_loop(
        0, _MAX_DET, topk_body, carry0)

    x1T = cxT - wT * 0.5
    y1T = cyT - hT * 0.5
    x2T = cxT + wT * 0.5
    y2T = cyT + hT * 0.5
    areaT = (x2T - x1T) * (y2T - y1T)

    def nms_body(i, keep):
        ohi = lT == i
        x1i = jnp.sum(jnp.where(ohi, x1T, 0.0))
        y1i = jnp.sum(jnp.where(ohi, y1T, 0.0))
        x2i = jnp.sum(jnp.where(ohi, x2T, 0.0))
        y2i = jnp.sum(jnp.where(ohi, y2T, 0.0))
        ari = jnp.sum(jnp.where(ohi, areaT, 0.0))
        ki = jnp.sum(jnp.where(ohi, keep, 0.0))
        iw = jnp.maximum(jnp.minimum(x2i, x2T) - jnp.maximum(x1i, x1T), 0.0)
        ih = jnp.maximum(jnp.minimum(y2i, y2T) - jnp.maximum(y1i, y1T), 0.0)
        inter = iw * ih
        iou = inter / (ari + areaT - inter + 1e-7)
        sup = (iou > _NMS_IOU) & (ki > 0.5) & (lT > i)
        return keep * jnp.where(sup, 0.0, 1.0)

    keep = lax.fori_loop(0, _MAX_DET, nms_body, jnp.ones((1, _NK), jnp.float32))

    valid = (keep > 0.5) & (cfT > _CONF_THRES)
    cfo = jnp.where(valid, cfT, 0.0)

    ri = lax.broadcasted_iota(jnp.int32, (_NK, _NK), 0)
    cj = lax.broadcasted_iota(jnp.int32, (_NK, _NK), 1)
    eye = ri == cj

    def tocol(row):
        return jnp.sum(jnp.where(eye, jnp.broadcast_to(row, (_NK, _NK)), 0.0),
                       axis=1, keepdims=True)

    cols = [tocol(v) for v in (cxT, cyT, wT, hT, cfo, clT)]
    l6 = lax.broadcasted_iota(jnp.int32, (_NK, 6), 1)
    out = jnp.zeros((_NK, 6), jnp.float32)
    for k in range(6):
        out = out + jnp.where(l6 == k, cols[k], 0.0)
    o_ref[0] = out[:_MAX_DET, :]


def kernel(raw):
    B, C, A = raw.shape
    pad = _APAD - A
    boxes = raw[:, :4, :]
    scores = raw[:, 4:, :]
    boxes_p = jnp.pad(boxes, ((0, 0), (0, 0), (0, pad)))
    scores_p = jnp.pad(scores, ((0, 0), (0, 0), (0, pad)), constant_values=-1.0)
    xp = jnp.concatenate([boxes_p, scores_p], axis=1).reshape(B, C, _ROWS, _LANES)

    return pl.pallas_call(
        _detect_kernel,
        grid=(B,),
        in_specs=[pl.BlockSpec((1, C, _ROWS, _LANES), lambda b: (b, 0, 0, 0))],
        out_specs=pl.BlockSpec((1, _MAX_DET, 6), lambda b: (b, 0, 0)),
        out_shape=jax.ShapeDtypeStruct((B, _MAX_DET, 6), jnp.float32),
        scratch_shapes=[pltpu.VMEM((_ROWS, _LANES), jnp.float32)],
    )(xp)
